# baseline (speedup 1.0000x reference)
import functools

import jax
import jax.numpy as jnp
from jax import lax
from jax.experimental import pallas as pl
from jax.experimental.pallas import tpu as pltpu
from jax.experimental.pallas import tpu_sc as plsc

_D = 768
_NP = 576
_NT = _NP + 1
_B = 64
_NC, _NS = 2, 16
_L = 16
_C = 16
_K = _NP // _C
_CPR = _D // _L


@functools.partial(
    pl.kernel,
    out_type=jax.ShapeDtypeStruct((_B, _NT, _D), jnp.float32),
    mesh=plsc.VectorSubcoreMesh(core_axis_name="c", subcore_axis_name="s"),
    scratch_types=[
        pltpu.VMEM((_C, _D), jnp.float32),
        pltpu.VMEM((_C, _D), jnp.float32),
        pltpu.VMEM((_C, _D), jnp.float32),
        pltpu.VMEM((_C, _D), jnp.float32),
        pltpu.VMEM((_C, _D), jnp.float32),
        pltpu.VMEM((_C, _D), jnp.float32),
        pltpu.VMEM((1, _D), jnp.float32),
        pltpu.VMEM((1, _D), jnp.float32),
        pltpu.VMEM((8, _D), jnp.float32),
        pltpu.VMEM((8, _D), jnp.float32),
        pltpu.SemaphoreType.DMA,
        pltpu.SemaphoreType.DMA,
        pltpu.SemaphoreType.DMA,
        pltpu.SemaphoreType.DMA,
        pltpu.SemaphoreType.DMA,
        pltpu.SemaphoreType.DMA,
        pltpu.SemaphoreType.DMA,
        pltpu.SemaphoreType.DMA,
        pltpu.SemaphoreType.DMA,
        pltpu.SemaphoreType.DMA,
    ],
)
def _sc_kernel(in_hbm, cls_hbm, pos_hbm, out_hbm,
               ac0, ac1, ac2, ac3, inb0, inb1,
               cls_v, carry_v, ptail_v, otail_v,
               ps0, ps1, ps2, ps3, is0, is1, os0, os1, os2, os3):
    wid = lax.axis_index("s") * _NC + lax.axis_index("c")
    accs = (ac0, ac1, ac2, ac3)
    po_sems = (ps0, ps1, ps2, ps3)
    out_sems = (os0, os1, os2, os3)
    in_bufs = (inb0, inb1)
    in_sems = (is0, is1)

    pltpu.sync_copy(cls_hbm, cls_v)
    t0 = pl.multiple_of(_NP + (wid - wid), 8)
    pltpu.sync_copy(pos_hbm.at[pl.ds(t0, 8)], ptail_v)

    def in_copy(s, b, k):
        r0 = pl.multiple_of(_C * k, 8)
        return pltpu.make_async_copy(
            in_hbm.at[b, pl.ds(r0, _C), :], in_bufs[s], in_sems[s])

    def pos_copy(j, k):
        r0 = pl.multiple_of(_C * k, 8)
        return pltpu.make_async_copy(pos_hbm.at[pl.ds(r0, _C)], accs[j], po_sems[j])

    def out_copy(j, b, k):
        r0 = pl.multiple_of(_C * k, 8)
        return pltpu.make_async_copy(
            accs[j], out_hbm.at[b, pl.ds(r0, _C), :], out_sems[j])

    for o in (0, 1):
        b = wid * 2 + o

        in_copy(0, b, 0).start()
        pos_copy(0, 0).start()
        in_copy(1, b, 1).start()
        pos_copy(1, 1).start()

        def quad_body(i, carry, b=b):
            for j in (0, 1, 2, 3):
                k = 4 * i + j
                s = j % 2
                in_copy(s, b, k).wait()
                pos_copy(j, k).wait()

                if j == 0:
                    @pl.when(i == 0)
                    def _():
                        for c in range(_CPR):
                            d = pl.ds(c * _L, _L)
                            accs[0][0, d] = accs[0][0, d] + cls_v[0, d]

                    @pl.when(i > 0)
                    def _():
                        for c in range(_CPR):
                            d = pl.ds(c * _L, _L)
                            accs[0][0, d] = accs[0][0, d] + carry_v[0, d]
                else:
                    for c in range(_CPR):
                        d = pl.ds(c * _L, _L)
                        accs[j][0, d] = accs[j][0, d] + carry_v[0, d]

                def row_body(r, rc):
                    for c in range(_CPR):
                        d = pl.ds(c * _L, _L)
                        accs[j][r, d] = accs[j][r, d] + in_bufs[s][r - 1, d]
                    return rc

                lax.fori_loop(1, _C, row_body, 0)

                for c in range(_CPR):
                    d = pl.ds(c * _L, _L)
                    carry_v[0, d] = in_bufs[s][_C - 1, d]

                out_copy(j, b, k).start()

                @pl.when(k <= _K - 3)
                def _():
                    @pl.when(k >= 2)
                    def _():
                        out_copy((j + 2) % 4, b, k - 2).wait()

                    in_copy(s, b, k + 2).start()
                    pos_copy((j + 2) % 4, k + 2).start()
            return carry

        lax.fori_loop(0, _K // 4, quad_body, 0)

        for c in range(_CPR):
            d = pl.ds(c * _L, _L)
            otail_v[0, d] = carry_v[0, d] + ptail_v[0, d]
        pltpu.sync_copy(otail_v, out_hbm.at[b, pl.ds(t0, 8), :])

        out_copy(0, b, _K - 4).wait()
        out_copy(1, b, _K - 3).wait()
        out_copy(2, b, _K - 2).wait()
        out_copy(3, b, _K - 1).wait()


def kernel(inputs, class_embed, pos_table):
    return _sc_kernel(inputs, class_embed.reshape(1, _D), pos_table)
